# baseline (speedup 1.0000x reference)
import functools

import jax
import jax.numpy as jnp
from jax import lax
from jax.experimental import pallas as pl
from jax.experimental.pallas import tpu as pltpu
from jax.experimental.pallas import tpu_sc as plsc

NW = 32
CH = 16


def _router_body(x_ref, rw_ref, out_ref, base_ref):
    step = pl.program_id(0)
    BTr, E = x_ref.shape[0], rw_ref.shape[0]

    @pl.when(step == 0)
    def _():
        base_ref[...] = jnp.zeros_like(base_ref)

    xb = x_ref[...]
    rw = rw_ref[...]
    logits = lax.dot_general(
        xb.astype(jnp.bfloat16), rw.astype(jnp.bfloat16),
        (((1,), (1,)), ((), ())),
        preferred_element_type=jnp.float32)
    lane = lax.broadcasted_iota(jnp.int32, (BTr, E), 1)
    m1 = jnp.max(logits, axis=1, keepdims=True)
    i1 = jnp.min(jnp.where(logits == m1, lane, E), axis=1, keepdims=True)
    masked = jnp.where(lane == i1, -jnp.inf, logits)
    m2 = jnp.max(masked, axis=1, keepdims=True)
    i2 = jnp.min(jnp.where(masked == m2, lane, E), axis=1, keepdims=True)
    s1 = jax.nn.sigmoid(m1)
    s2 = jax.nn.sigmoid(m2)
    den = s1 + s2 + 1e-10
    g1 = s1 / den
    g2 = s2 / den

    onehot = ((lane == i1) | (lane == i2)).astype(jnp.float32)
    r0 = lax.broadcasted_iota(jnp.int32, (BTr, BTr), 0)
    c0 = lax.broadcasted_iota(jnp.int32, (BTr, BTr), 1)
    tri = (r0 > c0).astype(jnp.float32)
    excl = lax.dot_general(
        tri, onehot, (((1,), (0,)), ((), ())),
        precision=lax.Precision.HIGHEST,
        preferred_element_type=jnp.float32)
    base = base_ref[...]
    rank_all = excl + base
    r1 = jnp.sum(jnp.where(lane == i1, rank_all, 0.0), axis=1, keepdims=True)
    r2 = jnp.sum(jnp.where(lane == i2, rank_all, 0.0), axis=1, keepdims=True)
    base_ref[...] = base + jnp.sum(onehot, axis=0, keepdims=True)

    W = out_ref.shape[1]
    lw = lax.broadcasted_iota(jnp.int32, (BTr, W), 1)
    out_ref[...] = (jnp.where(lw == 0, i1.astype(jnp.float32), 0.0)
                    + jnp.where(lw == 1, i2.astype(jnp.float32), 0.0)
                    + jnp.where(lw == 2, r1, 0.0)
                    + jnp.where(lw == 3, r2, 0.0)
                    + jnp.where(lw == 4, g1, 0.0)
                    + jnp.where(lw == 5, g2, 0.0))


def _router_tc(x, router_W, BTr):
    T, D = x.shape
    E = router_W.shape[0]
    return pl.pallas_call(
        _router_body,
        grid=(T // BTr,),
        in_specs=[
            pl.BlockSpec((BTr, D), lambda i: (i, 0)),
            pl.BlockSpec((E, D), lambda i: (0, 0)),
        ],
        out_specs=pl.BlockSpec((BTr, 128), lambda i: (i, 0)),
        out_shape=jax.ShapeDtypeStruct((T, 128), jnp.float32),
        scratch_shapes=[pltpu.VMEM((1, E), jnp.float32)],
    )(x, router_W)


def _dispatch_sc(x, dA, dB, NP):
    T, D = x.shape
    TW = T // NW
    nchunks = TW // CH
    dA3 = dA.reshape(NW, nchunks, CH)
    dB3 = dB.reshape(NW, nchunks, CH)
    mesh = plsc.VectorSubcoreMesh(core_axis_name="c", subcore_axis_name="s")

    @functools.partial(
        pl.kernel,
        out_type=jax.ShapeDtypeStruct((NP, D), x.dtype),
        mesh=mesh,
        scratch_types=[
            pltpu.VMEM((CH, D), x.dtype),
            pltpu.VMEM((CH,), jnp.int32),
            pltpu.VMEM((CH,), jnp.int32),
        ],
    )
    def k(x_hbm, dA_hbm, dB_hbm, xs_hbm, xbuf, idxA, idxB):
        w = lax.axis_index("s") * 2 + lax.axis_index("c")

        @pl.loop(0, nchunks)
        def _(c):
            base = w * TW + c * CH
            pltpu.sync_copy(x_hbm.at[pl.ds(base, CH)], xbuf)
            pltpu.sync_copy(dA_hbm.at[w, c], idxA)
            pltpu.sync_copy(dB_hbm.at[w, c], idxB)
            pltpu.sync_copy(xbuf, xs_hbm.at[idxA])
            pltpu.sync_copy(xbuf, xs_hbm.at[idxB])

    return k(x, dA3, dB3)


def _c1_body(te_ref, nt_ref, xs_ref, w1_ref, w3_ref, a_ref):
    i = pl.program_id(1)

    @pl.when(i < nt_ref[0])
    def _():
        xb = xs_ref[...]
        h = lax.dot_general(xb, w1_ref[0].astype(jnp.bfloat16),
                            (((1,), (1,)), ((), ())),
                            preferred_element_type=jnp.float32)
        g = lax.dot_general(xb, w3_ref[0].astype(jnp.bfloat16),
                            (((1,), (1,)), ((), ())),
                            preferred_element_type=jnp.float32)
        a_ref[...] = (h * jax.nn.sigmoid(h) * g).astype(a_ref.dtype)


def _c1_tc(te, nt, xs, W1, W3, BT, Fc, MT):
    NP, _ = xs.shape
    E, F, D = W1.shape
    NF = F // Fc
    grid_spec = pltpu.PrefetchScalarGridSpec(
        num_scalar_prefetch=2,
        grid=(NF, MT),
        in_specs=[
            pl.BlockSpec((BT, D),
                         lambda j, i, te, nt: (jnp.where(i < nt[0], i, 0), 0)),
            pl.BlockSpec((1, Fc, D), lambda j, i, te, nt: (te[i], j, 0)),
            pl.BlockSpec((1, Fc, D), lambda j, i, te, nt: (te[i], j, 0)),
        ],
        out_specs=pl.BlockSpec((BT, Fc), lambda j, i, te, nt: (i, j)),
    )
    return pl.pallas_call(
        _c1_body,
        grid_spec=grid_spec,
        out_shape=jax.ShapeDtypeStruct((NP, F), jnp.bfloat16),
    )(te, nt, xs, W1, W3)


def _c2_body(te_ref, nt_ref, a_ref, w2_ref, ys_ref):
    i = pl.program_id(1)

    @pl.when(i < nt_ref[0])
    def _():
        y = lax.dot_general(a_ref[...],
                            w2_ref[0].astype(jnp.bfloat16),
                            (((1,), (1,)), ((), ())),
                            preferred_element_type=jnp.float32)
        ys_ref[...] = y.astype(ys_ref.dtype)


def _c2_tc(te, nt, a, W2, BT, Dc, MT):
    NP, F = a.shape
    E, D, _ = W2.shape
    ND = D // Dc
    grid_spec = pltpu.PrefetchScalarGridSpec(
        num_scalar_prefetch=2,
        grid=(ND, MT),
        in_specs=[
            pl.BlockSpec((BT, F),
                         lambda j, i, te, nt: (jnp.where(i < nt[0], i, 0), 0)),
            pl.BlockSpec((1, Dc, F), lambda j, i, te, nt: (te[i], j, 0)),
        ],
        out_specs=pl.BlockSpec((BT, Dc), lambda j, i, te, nt: (i, j)),
    )
    return pl.pallas_call(
        _c2_body,
        grid_spec=grid_spec,
        out_shape=jax.ShapeDtypeStruct((NP, D), jnp.float32),
    )(te, nt, a, W2)


def _combine_sc(ys, dA, dB, gA16, gB16, T, D, out_dtype):
    TW = T // NW
    nchunks = TW // CH
    dA3 = dA.reshape(NW, nchunks, CH)
    dB3 = dB.reshape(NW, nchunks, CH)
    gA4 = gA16.reshape(NW, nchunks, CH, 16)
    gB4 = gB16.reshape(NW, nchunks, CH, 16)
    mesh = plsc.VectorSubcoreMesh(core_axis_name="c", subcore_axis_name="s")

    @functools.partial(
        pl.kernel,
        out_type=jax.ShapeDtypeStruct((T, D), out_dtype),
        mesh=mesh,
        scratch_types=[
            pltpu.VMEM((CH, D), jnp.float32),
            pltpu.VMEM((CH, D), jnp.float32),
            pltpu.VMEM((CH,), jnp.int32),
            pltpu.VMEM((CH,), jnp.int32),
            pltpu.VMEM((CH, 16), jnp.float32),
            pltpu.VMEM((CH, 16), jnp.float32),
        ],
    )
    def k(ys_hbm, dA_hbm, dB_hbm, gA_hbm, gB_hbm, out_hbm,
          bufA, bufB, idxA, idxB, gbA, gbB):
        w = lax.axis_index("s") * 2 + lax.axis_index("c")

        @pl.loop(0, nchunks)
        def _(c):
            base = w * TW + c * CH
            pltpu.sync_copy(dA_hbm.at[w, c], idxA)
            pltpu.sync_copy(dB_hbm.at[w, c], idxB)
            pltpu.sync_copy(gA_hbm.at[w, c], gbA)
            pltpu.sync_copy(gB_hbm.at[w, c], gbB)
            pltpu.sync_copy(ys_hbm.at[idxA], bufA)
            pltpu.sync_copy(ys_hbm.at[idxB], bufB)
            for r in range(CH):
                ga = gbA[r, :]
                gb = gbB[r, :]

                @pl.loop(0, D, step=16)
                def _(c1):
                    sl = (r, pl.ds(c1, 16))
                    bufA.at[*sl][...] = (bufA.at[*sl][...] * ga
                                         + bufB.at[*sl][...] * gb)
            pltpu.sync_copy(bufA, out_hbm.at[pl.ds(base, CH)])

    return k(ys, dA3, dB3, gA4, gB4)


def kernel(x, router_W, W1, W3, W2):
    T, D = x.shape
    E, F, _ = W1.shape
    K = 2
    BTr = 256
    BT = 256
    MT = (T * K) // BT + E - 1
    NP = MT * BT
    Fc = 1024
    Dc = 1024

    r = _router_tc(x, router_W, BTr)
    idx = r[:, 0:2].astype(jnp.int32)
    rank = r[:, 2:4].astype(jnp.int32)
    gates = r[:, 4:6]

    e_flat = idx.reshape(-1)
    r_flat = rank.reshape(-1)
    counts = jnp.sum(e_flat[:, None] == jnp.arange(E)[None, :], axis=0)
    ntiles = (counts + BT - 1) // BT
    tile_cum = jnp.cumsum(ntiles)
    num_tiles = tile_cum[-1]
    offs_pad = (tile_cum - ntiles) * BT
    dest = (offs_pad[e_flat] + r_flat).astype(jnp.int32)
    te = jnp.minimum(
        jnp.searchsorted(tile_cum, jnp.arange(MT), side='right'), E - 1
    ).astype(jnp.int32)
    nt = num_tiles.astype(jnp.int32).reshape(1)

    dA = dest[0::2]
    dB = dest[1::2]

    x_pk = lax.bitcast_convert_type(
        x.astype(jnp.bfloat16).reshape(T, D // 2, 2), jnp.int32)
    xs_pk = _dispatch_sc(x_pk, dA, dB, NP)
    xs = lax.bitcast_convert_type(xs_pk, jnp.bfloat16).reshape(NP, D)
    a = _c1_tc(te, nt, xs, W1, W3, BT, Fc, MT)
    ys = _c2_tc(te, nt, a, W2, BT, Dc, MT)

    gA16 = jnp.broadcast_to(gates[:, 0:1], (T, 16))
    gB16 = jnp.broadcast_to(gates[:, 1:2], (T, 16))
    out = _combine_sc(ys, dA, dB, gA16, gB16, T, D, x.dtype)
    return out


# speedup vs baseline: 1.4490x; 1.4490x over previous
import functools

import jax
import jax.numpy as jnp
from jax import lax
from jax.experimental import pallas as pl
from jax.experimental.pallas import tpu as pltpu
from jax.experimental.pallas import tpu_sc as plsc

NW = 32
CH = 16


def _router_body(x_ref, rw_ref, out_ref, base_ref):
    step = pl.program_id(0)
    BTr, E = x_ref.shape[0], rw_ref.shape[0]

    @pl.when(step == 0)
    def _():
        base_ref[...] = jnp.zeros_like(base_ref)

    xb = x_ref[...]
    rw = rw_ref[...]
    logits = lax.dot_general(
        xb.astype(jnp.bfloat16), rw.astype(jnp.bfloat16),
        (((1,), (1,)), ((), ())),
        preferred_element_type=jnp.float32)
    lane = lax.broadcasted_iota(jnp.int32, (BTr, E), 1)
    m1 = jnp.max(logits, axis=1, keepdims=True)
    i1 = jnp.min(jnp.where(logits == m1, lane, E), axis=1, keepdims=True)
    masked = jnp.where(lane == i1, -jnp.inf, logits)
    m2 = jnp.max(masked, axis=1, keepdims=True)
    i2 = jnp.min(jnp.where(masked == m2, lane, E), axis=1, keepdims=True)
    s1 = jax.nn.sigmoid(m1)
    s2 = jax.nn.sigmoid(m2)
    den = s1 + s2 + 1e-10
    g1 = s1 / den
    g2 = s2 / den

    onehot = ((lane == i1) | (lane == i2)).astype(jnp.float32)
    r0 = lax.broadcasted_iota(jnp.int32, (BTr, BTr), 0)
    c0 = lax.broadcasted_iota(jnp.int32, (BTr, BTr), 1)
    tri = (r0 > c0).astype(jnp.float32)
    excl = lax.dot_general(
        tri, onehot, (((1,), (0,)), ((), ())),
        precision=lax.Precision.HIGHEST,
        preferred_element_type=jnp.float32)
    base = base_ref[...]
    rank_all = excl + base
    r1 = jnp.sum(jnp.where(lane == i1, rank_all, 0.0), axis=1, keepdims=True)
    r2 = jnp.sum(jnp.where(lane == i2, rank_all, 0.0), axis=1, keepdims=True)
    base_ref[...] = base + jnp.sum(onehot, axis=0, keepdims=True)

    W = out_ref.shape[1]
    lw = lax.broadcasted_iota(jnp.int32, (BTr, W), 1)
    out_ref[...] = (jnp.where(lw == 0, i1.astype(jnp.float32), 0.0)
                    + jnp.where(lw == 1, i2.astype(jnp.float32), 0.0)
                    + jnp.where(lw == 2, r1, 0.0)
                    + jnp.where(lw == 3, r2, 0.0)
                    + jnp.where(lw == 4, g1, 0.0)
                    + jnp.where(lw == 5, g2, 0.0))


def _router_tc(x, router_W, BTr):
    T, D = x.shape
    E = router_W.shape[0]
    return pl.pallas_call(
        _router_body,
        grid=(T // BTr,),
        in_specs=[
            pl.BlockSpec((BTr, D), lambda i: (i, 0)),
            pl.BlockSpec((E, D), lambda i: (0, 0)),
        ],
        out_specs=pl.BlockSpec((BTr, 128), lambda i: (i, 0)),
        out_shape=jax.ShapeDtypeStruct((T, 128), jnp.float32),
        scratch_shapes=[pltpu.VMEM((1, E), jnp.float32)],
    )(x, router_W)


def _dispatch_sc(x, dA, dB, NP):
    T, D = x.shape
    TW = T // NW
    nchunks = TW // CH
    dA3 = dA.reshape(NW, nchunks, CH)
    dB3 = dB.reshape(NW, nchunks, CH)
    mesh = plsc.VectorSubcoreMesh(core_axis_name="c", subcore_axis_name="s")

    @functools.partial(
        pl.kernel,
        out_type=jax.ShapeDtypeStruct((NP, D), x.dtype),
        mesh=mesh,
        scratch_types=[
            pltpu.VMEM((CH, D), x.dtype),
            pltpu.VMEM((CH,), jnp.int32),
            pltpu.VMEM((CH,), jnp.int32),
        ],
    )
    def k(x_hbm, dA_hbm, dB_hbm, xs_hbm, xbuf, idxA, idxB):
        w = lax.axis_index("s") * 2 + lax.axis_index("c")

        @pl.loop(0, nchunks)
        def _(c):
            base = w * TW + c * CH
            pltpu.sync_copy(x_hbm.at[pl.ds(base, CH)], xbuf)
            pltpu.sync_copy(dA_hbm.at[w, c], idxA)
            pltpu.sync_copy(dB_hbm.at[w, c], idxB)
            pltpu.sync_copy(xbuf, xs_hbm.at[idxA])
            pltpu.sync_copy(xbuf, xs_hbm.at[idxB])

    return k(x, dA3, dB3)


def _c1_body(te_ref, nt_ref, xs_ref, w1_ref, w3_ref, a_ref):
    i = pl.program_id(1)

    @pl.when(i < nt_ref[0])
    def _():
        xb = xs_ref[...].astype(jnp.bfloat16)
        h = lax.dot_general(xb, w1_ref[0].astype(jnp.bfloat16),
                            (((1,), (1,)), ((), ())),
                            preferred_element_type=jnp.float32)
        g = lax.dot_general(xb, w3_ref[0].astype(jnp.bfloat16),
                            (((1,), (1,)), ((), ())),
                            preferred_element_type=jnp.float32)
        a_ref[...] = (h * jax.nn.sigmoid(h) * g).astype(a_ref.dtype)


def _c1_tc(te, nt, xs, W1, W3, BT, Fc, MT):
    NP, _ = xs.shape
    E, F, D = W1.shape
    NF = F // Fc
    grid_spec = pltpu.PrefetchScalarGridSpec(
        num_scalar_prefetch=2,
        grid=(NF, MT),
        in_specs=[
            pl.BlockSpec((BT, D),
                         lambda j, i, te, nt: (jnp.where(i < nt[0], i, 0), 0)),
            pl.BlockSpec((1, Fc, D), lambda j, i, te, nt: (te[i], j, 0)),
            pl.BlockSpec((1, Fc, D), lambda j, i, te, nt: (te[i], j, 0)),
        ],
        out_specs=pl.BlockSpec((BT, Fc), lambda j, i, te, nt: (i, j)),
    )
    return pl.pallas_call(
        _c1_body,
        grid_spec=grid_spec,
        out_shape=jax.ShapeDtypeStruct((NP, F), jnp.bfloat16),
    )(te, nt, xs, W1, W3)


def _c2_body(te_ref, nt_ref, a_ref, w2_ref, ys_ref):
    i = pl.program_id(1)

    @pl.when(i < nt_ref[0])
    def _():
        y = lax.dot_general(a_ref[...],
                            w2_ref[0].astype(jnp.bfloat16),
                            (((1,), (1,)), ((), ())),
                            preferred_element_type=jnp.float32)
        ys_ref[...] = y.astype(ys_ref.dtype)


def _c2_tc(te, nt, a, W2, BT, Dc, MT):
    NP, F = a.shape
    E, D, _ = W2.shape
    ND = D // Dc
    grid_spec = pltpu.PrefetchScalarGridSpec(
        num_scalar_prefetch=2,
        grid=(ND, MT),
        in_specs=[
            pl.BlockSpec((BT, F),
                         lambda j, i, te, nt: (jnp.where(i < nt[0], i, 0), 0)),
            pl.BlockSpec((1, Dc, F), lambda j, i, te, nt: (te[i], j, 0)),
        ],
        out_specs=pl.BlockSpec((BT, Dc), lambda j, i, te, nt: (i, j)),
    )
    return pl.pallas_call(
        _c2_body,
        grid_spec=grid_spec,
        out_shape=jax.ShapeDtypeStruct((NP, D), jnp.float32),
    )(te, nt, a, W2)


def _combine_sc(ys, dA, dB, gA16, gB16, T, D, out_dtype):
    TW = T // NW
    nchunks = TW // CH
    dA3 = dA.reshape(NW, nchunks, CH)
    dB3 = dB.reshape(NW, nchunks, CH)
    gA4 = gA16.reshape(NW, nchunks, CH, 16)
    gB4 = gB16.reshape(NW, nchunks, CH, 16)
    mesh = plsc.VectorSubcoreMesh(core_axis_name="c", subcore_axis_name="s")

    @functools.partial(
        pl.kernel,
        out_type=jax.ShapeDtypeStruct((T, D), out_dtype),
        mesh=mesh,
        scratch_types=[
            pltpu.VMEM((CH, D), jnp.float32),
            pltpu.VMEM((CH, D), jnp.float32),
            pltpu.VMEM((CH,), jnp.int32),
            pltpu.VMEM((CH,), jnp.int32),
            pltpu.VMEM((CH, 16), jnp.float32),
            pltpu.VMEM((CH, 16), jnp.float32),
        ],
    )
    def k(ys_hbm, dA_hbm, dB_hbm, gA_hbm, gB_hbm, out_hbm,
          bufA, bufB, idxA, idxB, gbA, gbB):
        w = lax.axis_index("s") * 2 + lax.axis_index("c")

        @pl.loop(0, nchunks)
        def _(c):
            base = w * TW + c * CH
            pltpu.sync_copy(dA_hbm.at[w, c], idxA)
            pltpu.sync_copy(dB_hbm.at[w, c], idxB)
            pltpu.sync_copy(gA_hbm.at[w, c], gbA)
            pltpu.sync_copy(gB_hbm.at[w, c], gbB)
            pltpu.sync_copy(ys_hbm.at[idxA], bufA)
            pltpu.sync_copy(ys_hbm.at[idxB], bufB)
            for r in range(CH):
                ga = gbA[r, :]
                gb = gbB[r, :]

                @pl.loop(0, D, step=16)
                def _(c1):
                    sl = (r, pl.ds(c1, 16))
                    bufA.at[*sl][...] = (bufA.at[*sl][...] * ga
                                         + bufB.at[*sl][...] * gb)
            pltpu.sync_copy(bufA, out_hbm.at[pl.ds(base, CH)])

    return k(ys, dA3, dB3, gA4, gB4)


def kernel(x, router_W, W1, W3, W2):
    T, D = x.shape
    E, F, _ = W1.shape
    K = 2
    BTr = 256
    BT = 256
    MT = (T * K) // BT + E - 1
    NP = MT * BT
    Fc = 1024
    Dc = 1024

    r = _router_tc(x, router_W, BTr)
    idx = r[:, 0:2].astype(jnp.int32)
    rank = r[:, 2:4].astype(jnp.int32)
    gates = r[:, 4:6]

    e_flat = idx.reshape(-1)
    r_flat = rank.reshape(-1)
    counts = jnp.sum(e_flat[:, None] == jnp.arange(E)[None, :], axis=0)
    ntiles = (counts + BT - 1) // BT
    tile_cum = jnp.cumsum(ntiles)
    num_tiles = tile_cum[-1]
    offs_pad = (tile_cum - ntiles) * BT
    dest = (offs_pad[e_flat] + r_flat).astype(jnp.int32)
    te = jnp.minimum(
        jnp.searchsorted(tile_cum, jnp.arange(MT), side='right'), E - 1
    ).astype(jnp.int32)
    nt = num_tiles.astype(jnp.int32).reshape(1)

    dA = dest[0::2]
    dB = dest[1::2]

    xs = _dispatch_sc(x, dA, dB, NP)
    a = _c1_tc(te, nt, xs, W1, W3, BT, Fc, MT)
    ys = _c2_tc(te, nt, a, W2, BT, Dc, MT)

    gA16 = jnp.broadcast_to(gates[:, 0:1], (T, 16))
    gB16 = jnp.broadcast_to(gates[:, 1:2], (T, 16))
    out = _combine_sc(ys, dA, dB, gA16, gB16, T, D, x.dtype)
    return out
